# Initial kernel scaffold; baseline (speedup 1.0000x reference)
import jax
import jax.numpy as jnp
from jax.experimental import pallas as pl


def kernel(grid_x, mesh_x, m2m_edge_attr, g2m_edge_attr, g2m_src, g2m_dst, params):
    raise NotImplementedError("write your pallas kernel here")


# speedup vs baseline: 1.0368x; 1.0368x over previous
import functools

import jax
import jax.numpy as jnp
from jax import lax
from jax.experimental import pallas as pl
from jax.experimental.pallas import tpu as pltpu


LATENT = 64
EPS = 1e-5


def _pick_block(n, target):
    b = min(n, target)
    while b > 8 and (n % b or b % 8):
        b -= 8
    return b if n % b == 0 else n


def _mlp_ln(x, w1, b1, w2, b2, g, be):
    h = jnp.dot(x, w1, preferred_element_type=jnp.float32) + b1
    h = h * jax.nn.sigmoid(h)
    y = jnp.dot(h, w2, preferred_element_type=jnp.float32) + b2
    mu = jnp.mean(y, axis=-1, keepdims=True)
    var = jnp.mean((y - mu) ** 2, axis=-1, keepdims=True)
    return (y - mu) * lax.rsqrt(var + EPS) * g + be


def _p6(p):
    return (p["w1"], p["b1"].reshape(1, -1), p["w2"], p["b2"].reshape(1, -1),
            p["g"].reshape(1, -1), p["be"].reshape(1, -1))


def _full_spec(a):
    return pl.BlockSpec(a.shape, lambda i: (0,) * a.ndim)


def _row_spec(a, b):
    return pl.BlockSpec((b,) + a.shape[1:], lambda i: (i,) + (0,) * (a.ndim - 1))


def _grid_kernel(x_ref, *refs):
    (w1, b1, w2, b2, g, be,
     uw1, ub1, uw2, ub2, ug, ube, wes, ugx_ref, a_ref) = refs
    gx = _mlp_ln(x_ref[...], w1[...], b1[...], w2[...], b2[...], g[...], be[...])
    ugx_ref[...] = gx + _mlp_ln(gx, uw1[...], ub1[...], uw2[...], ub2[...],
                                ug[...], ube[...])
    a_ref[...] = jnp.dot(gx, wes[...], preferred_element_type=jnp.float32)


def _mesh_kernel(x_ref, *refs):
    w1, b1, w2, b2, g, be, wer, mx_ref, b_ref = refs
    mx = _mlp_ln(x_ref[...], w1[...], b1[...], w2[...], b2[...], g[...], be[...])
    mx_ref[...] = mx
    b_ref[...] = jnp.dot(mx, wer[...], preferred_element_type=jnp.float32)


def _m2m_kernel(x_ref, *refs):
    w1, b1, w2, b2, g, be, out_ref = refs
    out_ref[...] = _mlp_ln(x_ref[...], w1[...], b1[...], w2[...], b2[...],
                           g[...], be[...])


def _edge_kernel(attr_ref, as_ref, bd_ref, *refs):
    (w1, b1, w2, b2, g, be,
     wec, b1e, w2e, b2e, ge, bee, ug2m_ref, upd_ref) = refs
    emb = _mlp_ln(attr_ref[...], w1[...], b1[...], w2[...], b2[...],
                  g[...], be[...])
    pre = (as_ref[...] + bd_ref[...]
           + jnp.dot(emb, wec[...], preferred_element_type=jnp.float32)
           + b1e[...])
    h = pre * jax.nn.sigmoid(pre)
    y = jnp.dot(h, w2e[...], preferred_element_type=jnp.float32) + b2e[...]
    mu = jnp.mean(y, axis=-1, keepdims=True)
    var = jnp.mean((y - mu) ** 2, axis=-1, keepdims=True)
    upd = (y - mu) * lax.rsqrt(var + EPS) * ge[...] + bee[...]
    upd_ref[...] = upd
    ug2m_ref[...] = emb + upd


def _node_kernel(mx_ref, agg0_ref, agg1_ref, *refs):
    w1m, w1a, b1, w2, b2, g, be, out_ref = refs
    mx = mx_ref[...]
    agg = agg0_ref[...] + agg1_ref[...]
    pre = (jnp.dot(mx, w1m[...], preferred_element_type=jnp.float32)
           + jnp.dot(agg, w1a[...], preferred_element_type=jnp.float32)
           + b1[...])
    h = pre * jax.nn.sigmoid(pre)
    y = jnp.dot(h, w2[...], preferred_element_type=jnp.float32) + b2[...]
    mu = jnp.mean(y, axis=-1, keepdims=True)
    var = jnp.mean((y - mu) ** 2, axis=-1, keepdims=True)
    out_ref[...] = mx + (y - mu) * lax.rsqrt(var + EPS) * g[...] + be[...]


def _row_call(body, n, blk, row_ins, full_ins, n_out):
    grid = (n // blk,)
    out_shape = [jax.ShapeDtypeStruct((n, LATENT), jnp.float32)] * n_out
    out_specs = [pl.BlockSpec((blk, LATENT), lambda i: (i, 0))] * n_out
    return pl.pallas_call(
        body,
        grid=grid,
        in_specs=[_row_spec(a, blk) for a in row_ins]
        + [_full_spec(a) for a in full_ins],
        out_specs=out_specs,
        out_shape=out_shape,
    )(*row_ins, *full_ins)


def kernel(grid_x, mesh_x, m2m_edge_attr, g2m_edge_attr, g2m_src, g2m_dst, params):
    n_grid = grid_x.shape[0]
    n_mesh = mesh_x.shape[0]
    e_g2m = g2m_edge_attr.shape[0]
    e_m2m = m2m_edge_attr.shape[0]

    pe = params["edge_mlp"]
    w1e = pe["w1"]
    w1e_s, w1e_r, w1e_c = w1e[:LATENT], w1e[LATENT:2 * LATENT], w1e[2 * LATENT:]
    pn = params["node_mlp"]
    w1n_m, w1n_a = pn["w1"][:LATENT], pn["w1"][LATENT:]

    bg = _pick_block(n_grid, 2000)
    updated_gx, a_tab = _row_call(
        _grid_kernel, n_grid, bg, [grid_x],
        [*_p6(params["grid_embed"]), *_p6(params["grid_update"]), w1e_s], 2)

    bm = _pick_block(n_mesh, 2000)
    mx, b_tab = _row_call(
        _mesh_kernel, n_mesh, bm, [mesh_x],
        [*_p6(params["mesh_embed"]), w1e_r], 2)

    bmm = _pick_block(e_m2m, 4000)
    (m2m,) = _row_call(_m2m_kernel, e_m2m, bmm, [m2m_edge_attr],
                       [*_p6(params["m2m_edge_embed"])], 1)

    a_src = jnp.take(a_tab, g2m_src, axis=0)
    b_dst = jnp.take(b_tab, g2m_dst, axis=0)

    be_blk = _pick_block(e_g2m, 4000)
    updated_g2m, upd_edge = _row_call(
        _edge_kernel, e_g2m, be_blk, [g2m_edge_attr, a_src, b_dst],
        [*_p6(params["g2m_edge_embed"]),
         w1e_c, pe["b1"].reshape(1, -1), pe["w2"], pe["b2"].reshape(1, -1),
         pe["g"].reshape(1, -1), pe["be"].reshape(1, -1)], 2)

    agg = jax.ops.segment_sum(upd_edge, g2m_dst, num_segments=n_mesh)
    agg0, agg1 = agg, jnp.zeros_like(agg)

    bn = _pick_block(n_mesh, 2000)
    (updated_mx,) = _row_call(
        _node_kernel, n_mesh, bn, [mx, agg0, agg1],
        [w1n_m, w1n_a, pn["b1"].reshape(1, -1), pn["w2"],
         pn["b2"].reshape(1, -1), pn["g"].reshape(1, -1),
         pn["be"].reshape(1, -1)], 1)

    return (updated_gx, updated_mx, m2m, updated_g2m)
